# Initial kernel scaffold; baseline (speedup 1.0000x reference)
import jax
import jax.numpy as jnp
from jax.experimental import pallas as pl


def kernel(x, W_qkv, b_qkv, W_proj, b_proj, layer_idx, total_layers):
    raise NotImplementedError("write your pallas kernel here")


# speedup vs baseline: 8.8879x; 8.8879x over previous
import functools

import jax
import jax.numpy as jnp
from jax.experimental import pallas as pl
from jax.experimental.pallas import tpu as pltpu

_REF_PREC = jax.lax.Precision.DEFAULT
_EXACT_PREC = jax.lax.Precision.HIGHEST


def _qkv_select_body(x_ref, w_ref, b_ref, m_ref, qkv_ref, assign_ref, *,
                     T, C, K, NH):
    xb = x_ref[0]
    qkv = jnp.dot(xb, w_ref[...], preferred_element_type=jnp.float32,
                  precision=_REF_PREC)
    qkv = qkv + b_ref[...]
    qkv_ref[0] = qkv

    kpart = qkv[:, C:2 * C]
    hs = jnp.dot(kpart * kpart, m_ref[...],
                 preferred_element_type=jnp.float32,
                 precision=_EXACT_PREC)
    hmag = jnp.sqrt(hs)
    nrm = jnp.sqrt(jnp.sum(hmag * hmag, axis=1, keepdims=True))
    phi = hmag / jnp.clip(nrm, 1e-12, None)

    meanp = jnp.mean(phi, axis=0, keepdims=True)
    score = jnp.sum(phi * meanp, axis=1, keepdims=True)
    row_id = jax.lax.broadcasted_iota(jnp.int32, (T, 1), 0)
    score = jnp.where(row_id == 0, -jnp.inf, score)
    score_r = jnp.transpose(score)

    ii = jax.lax.broadcasted_iota(jnp.int32, (T, T), 0)
    jj = jax.lax.broadcasted_iota(jnp.int32, (T, T), 1)
    ltr = (score < score_r).astype(jnp.float32)
    eqr = ((score == score_r) & (ii < jj)).astype(jnp.float32)
    rank_row = jnp.sum(ltr + eqr, axis=0, keepdims=True)
    keep_row = rank_row < K

    upper = (ii <= jj).astype(jnp.float32)
    pos_row = jnp.dot(keep_row.astype(jnp.float32), upper,
                      preferred_element_type=jnp.float32) - 1.0

    sim = jax.lax.dot_general(phi, phi, (((1,), (1,)), ((), ())),
                              preferred_element_type=jnp.float32,
                              precision=_REF_PREC)
    simm = jnp.where(keep_row, sim, -jnp.inf)
    mx = jnp.max(simm, axis=1, keepdims=True)
    cand = jnp.where(simm == mx, pos_row, jnp.float32(1e9))
    assign_f = jnp.min(cand, axis=1, keepdims=True)
    assign_ref[0] = jnp.transpose(assign_f).astype(jnp.int32)


def _merge_attn_body(qkv_ref, assign_ref, wp_ref, bp_ref, out_ref, *,
                     T, C, H, HD, K, SCALE):
    qkvb = qkv_ref[0]
    a_row = assign_ref[0]
    kk = jax.lax.broadcasted_iota(jnp.int32, (K, T), 0)
    sel = (kk == a_row).astype(jnp.float32)
    mq = jnp.dot(sel, qkvb, preferred_element_type=jnp.float32,
                 precision=_EXACT_PREC)
    den = jnp.sum(sel, axis=1, keepdims=True)
    dclip = jnp.clip(den, 1e-12, None)

    ys = []
    for h in range(H):
        qh = mq[:, h * HD:(h + 1) * HD] / dclip * SCALE
        kh = mq[:, C + h * HD:C + (h + 1) * HD] / dclip
        vh = mq[:, 2 * C + h * HD:2 * C + (h + 1) * HD] / dclip
        lg = jax.lax.dot_general(qh, kh, (((1,), (1,)), ((), ())),
                                 preferred_element_type=jnp.float32,
                                 precision=_REF_PREC)
        mxl = jnp.max(lg, axis=1, keepdims=True)
        e = jnp.exp(lg - mxl)
        p = e / jnp.sum(e, axis=1, keepdims=True)
        ys.append(jnp.dot(p, vh, preferred_element_type=jnp.float32,
                          precision=_REF_PREC))
    y = jnp.concatenate(ys, axis=1)
    y = jnp.dot(y, wp_ref[...], preferred_element_type=jnp.float32,
                precision=_REF_PREC)
    y = y + bp_ref[...]
    out_ref[0] = jax.lax.dot_general(sel, y, (((0,), (0,)), ((), ())),
                                     preferred_element_type=jnp.float32,
                                     precision=_EXACT_PREC)


def _qkv_select_call(x, W_qkv, b_qkv, H, R):
    B, T, C = x.shape
    HD = C // H
    K = T - R
    NH = 128
    f32 = jnp.float32

    head_onehot = (jnp.arange(C)[:, None] // HD ==
                   jnp.arange(NH)[None, :]).astype(f32)
    b_qkv2 = b_qkv.reshape(1, 3 * C)

    return pl.pallas_call(
        functools.partial(_qkv_select_body, T=T, C=C, K=K, NH=NH),
        grid=(B,),
        in_specs=[
            pl.BlockSpec((1, T, C), lambda b: (b, 0, 0)),
            pl.BlockSpec((C, 3 * C), lambda b: (0, 0)),
            pl.BlockSpec((1, 3 * C), lambda b: (0, 0)),
            pl.BlockSpec((C, NH), lambda b: (0, 0)),
        ],
        out_specs=[
            pl.BlockSpec((1, T, 3 * C), lambda b: (b, 0, 0)),
            pl.BlockSpec((1, 1, T), lambda b: (b, 0, 0)),
        ],
        out_shape=[
            jax.ShapeDtypeStruct((B, T, 3 * C), f32),
            jax.ShapeDtypeStruct((B, 1, T), jnp.int32),
        ],
        compiler_params=pltpu.CompilerParams(
            dimension_semantics=("arbitrary",)),
    )(x, W_qkv, b_qkv2, head_onehot)


def _merge_attn_call(qkv, assign, W_proj, b_proj, H, R):
    B, T, _ = qkv.shape
    C = qkv.shape[2] // 3
    HD = C // H
    K = T - R
    SCALE = 1.0 / (HD ** 0.5)
    f32 = jnp.float32
    b_proj2 = b_proj.reshape(1, C)

    return pl.pallas_call(
        functools.partial(_merge_attn_body, T=T, C=C, H=H, HD=HD, K=K,
                          SCALE=SCALE),
        grid=(B,),
        in_specs=[
            pl.BlockSpec((1, T, 3 * C), lambda b: (b, 0, 0)),
            pl.BlockSpec((1, 1, T), lambda b: (b, 0, 0)),
            pl.BlockSpec((C, C), lambda b: (0, 0)),
            pl.BlockSpec((1, C), lambda b: (0, 0)),
        ],
        out_specs=pl.BlockSpec((1, T, C), lambda b: (b, 0, 0)),
        out_shape=jax.ShapeDtypeStruct((B, T, C), f32),
        compiler_params=pltpu.CompilerParams(
            dimension_semantics=("arbitrary",)),
    )(qkv, assign, W_proj, b_proj2)


def _run(x, W_qkv, b_qkv, W_proj, b_proj, H, R):
    qkv, assign = _qkv_select_call(x, W_qkv, b_qkv, H, R)
    return _merge_attn_call(qkv, assign, W_proj, b_proj, H, R)


def kernel(x, W_qkv, b_qkv, W_proj, b_proj, layer_idx, total_layers):
    return _run(x, W_qkv, b_qkv, W_proj, b_proj, H=12, R=128)
